# baseline (speedup 1.0000x reference)
import functools

import jax
import jax.numpy as jnp
from jax import lax
from jax.experimental import pallas as pl
from jax.experimental.pallas import tpu as pltpu
from jax.experimental.pallas import tpu_sc as plsc

N = 10000
E = 320000
DF = 128
C1 = 64
C2 = 128
NCLS = 10
NG = 32

NC = 2
NS = 16
L = 16
NW = NC * NS

CH = 128
CPW = 80
EPW = CH * CPW
E_PAD = EPW * NW
NPAD = 10240
RPT = NPAD // NS
TPAD = 10048

_MESH = plsc.VectorSubcoreMesh(core_axis_name="c", subcore_axis_name="s")


@functools.partial(
    pl.kernel,
    out_type=jax.ShapeDtypeStruct((NC, NPAD), jnp.float32),
    mesh=_MESH,
    scratch_types=[
        pltpu.VMEM((CPW, CH), jnp.int32),
        pltpu.VMEM((CH,), jnp.float32),
        pltpu.VMEM((RPT,), jnp.float32),
        pltpu.VMEM_SHARED((NPAD,), jnp.float32),
    ],
)
def _sc_degree(dst_hbm, out_hbm, idx_v, ones_v, z_v, acc):
    cid = lax.axis_index("c")
    sid = lax.axis_index("s")
    wid = sid * NC + cid
    pltpu.sync_copy(dst_hbm.at[pl.ds(wid * CPW, CPW)], idx_v)
    for i in range(CH // L):
        ones_v[pl.ds(i * L, L)] = jnp.ones((L,), jnp.float32)

    def zfill(i, c):
        z_v[pl.ds(i * L, L)] = jnp.zeros((L,), jnp.float32)
        return c

    lax.fori_loop(0, RPT // L, zfill, 0)
    pltpu.sync_copy(z_v, acc.at[pl.ds(sid * RPT, RPT)])
    plsc.subcore_barrier()

    def body(j, c):
        pltpu.sync_copy(ones_v, acc.at[idx_v.at[j]], add=True)
        return c

    lax.fori_loop(0, CPW, body, 0)
    plsc.subcore_barrier()
    pltpu.sync_copy(acc.at[pl.ds(sid * RPT, RPT)],
                    out_hbm.at[cid, pl.ds(sid * RPT, RPT)])


D = 128
HCPW = CPW // 2


@functools.partial(
    pl.kernel,
    out_type=jax.ShapeDtypeStruct((NC, NPAD, D), jnp.float32),
    mesh=_MESH,
    scratch_types=[
        pltpu.VMEM((HCPW, CH), jnp.int32),
        pltpu.VMEM((HCPW, CH), jnp.int32),
        pltpu.VMEM((CH, D), jnp.float32),
        pltpu.VMEM((CH, D), jnp.float32),
        pltpu.VMEM_SHARED((NPAD, D), jnp.float32),
        pltpu.SemaphoreType.DMA,
        pltpu.SemaphoreType.DMA,
    ],
)
def _sc_agg(table_hbm, src_hbm, dst_hbm, out_hbm, src_v, dst_v,
            rows_a, rows_b, acc, sem_a, sem_b):
    cid = lax.axis_index("c")
    sid = lax.axis_index("s")
    wid = sid * NC + cid

    def zfill(i, c):
        r = i // (D // L)
        k = i % (D // L)
        rows_a[r, pl.ds(k * L, L)] = jnp.zeros((L,), jnp.float32)
        return c

    with jax.named_scope("agg_zero"):
        lax.fori_loop(0, CH * (D // L), zfill, 0)
        for b in range(RPT // CH):
            pltpu.sync_copy(rows_a, acc.at[pl.ds(sid * RPT + b * CH, CH)])
        plsc.subcore_barrier()

    for h in range(CPW // HCPW):
        with jax.named_scope("agg_idx"):
            base = wid * CPW + h * HCPW
            pltpu.sync_copy(src_hbm.at[pl.ds(base, HCPW)], src_v)
            pltpu.sync_copy(dst_hbm.at[pl.ds(base, HCPW)], dst_v)
        pltpu.async_copy(table_hbm.at[src_v.at[0]], rows_a, sem_a)
        pltpu.async_copy(table_hbm.at[src_v.at[1]], rows_b, sem_b)

        def body(i, c):
            ja = 2 * i
            jb = ja + 1
            pltpu.make_async_copy(table_hbm, rows_a, sem_a).wait()
            pltpu.sync_copy(rows_a, acc.at[dst_v.at[ja]], add=True)

            @pl.when(ja + 2 < HCPW)
            def _():
                pltpu.async_copy(table_hbm.at[src_v.at[ja + 2]], rows_a, sem_a)

            pltpu.make_async_copy(table_hbm, rows_b, sem_b).wait()
            pltpu.sync_copy(rows_b, acc.at[dst_v.at[jb]], add=True)

            @pl.when(jb + 2 < HCPW)
            def _():
                pltpu.async_copy(table_hbm.at[src_v.at[jb + 2]], rows_b, sem_b)

            return c

        with jax.named_scope("agg_loop"):
            lax.fori_loop(0, HCPW // 2, body, 0)

    with jax.named_scope("agg_dump"):
        plsc.subcore_barrier()
        pltpu.sync_copy(acc.at[pl.ds(sid * RPT, RPT)],
                        out_hbm.at[cid, pl.ds(sid * RPT, RPT)])


def _tc_prescale_body(x_ref, w_ref, d0_ref, d1_ref, hs_ref, dinv_ref):
    deg = d0_ref[...] + d1_ref[...] + 1.0
    dinv = lax.rsqrt(deg)
    dinv_ref[...] = dinv
    h = jnp.dot(x_ref[...], w_ref[...], preferred_element_type=jnp.float32)
    hs_ref[0:N, 0:C1] = h * dinv
    hs_ref[0:N, C1:D] = jnp.zeros((N, D - C1), jnp.float32)
    hs_ref[N:TPAD, :] = jnp.zeros((TPAD - N, D), jnp.float32)


def _tc_layer2_body(a0_ref, a1_ref, hs_ref, dinv_ref, b1_ref, w2_ref,
                    h2s_ref):
    dinv = dinv_ref[...]
    pre = dinv * (a0_ref[...] + a1_ref[...] + hs_ref[0:N, 0:C1]) + b1_ref[...]
    h1 = jnp.maximum(pre, 0.0)
    h2 = jnp.dot(h1, w2_ref[...], preferred_element_type=jnp.float32)
    h2s_ref[0:N, :] = h2 * dinv
    h2s_ref[N:TPAD, :] = jnp.zeros((TPAD - N, C2), jnp.float32)


def _tc_head_body(a0_ref, a1_ref, hs_ref, dinv_ref, b2_ref, batch_ref,
                  fcw_ref, fcb_ref, out_ref):
    dinv = dinv_ref[...]
    pre = dinv * (a0_ref[...] + a1_ref[...] + hs_ref[0:N, :]) + b2_ref[...]
    h2 = jnp.maximum(pre, 0.0)
    gid = lax.broadcasted_iota(jnp.int32, (N, NG), 1)
    p = (batch_ref[...] == gid).astype(jnp.float32)
    sums = lax.dot_general(p, h2, (((0,), (0,)), ((), ())),
                           preferred_element_type=jnp.float32)
    ones = jnp.ones((N, 1), jnp.float32)
    cnt = lax.dot_general(p, ones, (((0,), (0,)), ((), ())),
                          preferred_element_type=jnp.float32)
    g = sums / jnp.maximum(cnt, 1.0)
    logits = jnp.dot(g, fcw_ref[...], preferred_element_type=jnp.float32)
    logits = logits + fcb_ref[...]
    m = jnp.max(logits, axis=1, keepdims=True)
    ex = jnp.exp(logits - m)
    lse = jnp.log(jnp.sum(ex, axis=1, keepdims=True))
    out_ref[...] = logits - m - lse


_tc_prescale = pl.pallas_call(
    _tc_prescale_body,
    out_shape=(jax.ShapeDtypeStruct((TPAD, D), jnp.float32),
               jax.ShapeDtypeStruct((N, 1), jnp.float32)))

_tc_layer2 = pl.pallas_call(
    _tc_layer2_body,
    out_shape=jax.ShapeDtypeStruct((TPAD, C2), jnp.float32))

_tc_head = pl.pallas_call(
    _tc_head_body,
    out_shape=jax.ShapeDtypeStruct((NG, NCLS), jnp.float32))


def kernel(x, edge_index, batch, W1, b1, W2, b2, fc_W, fc_b):
    src = edge_index[0].astype(jnp.int32)
    dst = edge_index[1].astype(jnp.int32)
    pad = jnp.full((E_PAD - E,), N, jnp.int32)
    src2d = jnp.concatenate([src, pad]).reshape(NW * CPW, CH)
    dst2d = jnp.concatenate([dst, pad]).reshape(NW * CPW, CH)
    batch2d = batch.astype(jnp.int32).reshape(N, 1)

    degp = _sc_degree(dst2d)
    d0 = degp[0, :N].reshape(N, 1)
    d1 = degp[1, :N].reshape(N, 1)

    h1s, dinv = _tc_prescale(x, W1, d0, d1)

    agg1 = _sc_agg(h1s, src2d, dst2d)
    h2s = _tc_layer2(agg1[0, :N, :C1], agg1[1, :N, :C1], h1s, dinv,
                     b1.reshape(1, C1), W2)

    agg2 = _sc_agg(h2s, src2d, dst2d)
    out = _tc_head(agg2[0, :N, :], agg2[1, :N, :], h2s, dinv,
                   b2.reshape(1, C2), batch2d,
                   fc_W, fc_b.reshape(1, NCLS))
    return out


# speedup vs baseline: 3.5507x; 3.5507x over previous
import functools

import jax
import jax.numpy as jnp
from jax import lax
from jax.experimental import pallas as pl
from jax.experimental.pallas import tpu as pltpu
from jax.experimental.pallas import tpu_sc as plsc

N = 10000
E = 320000
DF = 128
C1 = 64
C2 = 128
NCLS = 10
NG = 32

NC = 2
NS = 16
L = 16
NW = NC * NS

CH = 128
CPW = 80
EPW = CH * CPW
E_PAD = EPW * NW
NPAD = 10240
RPT = NPAD // NS
TPAD = 10048

_MESH = plsc.VectorSubcoreMesh(core_axis_name="c", subcore_axis_name="s")


@functools.partial(
    pl.kernel,
    out_type=jax.ShapeDtypeStruct((NC, NPAD), jnp.float32),
    mesh=_MESH,
    scratch_types=[
        pltpu.VMEM((CPW, CH), jnp.int32),
        pltpu.VMEM((CH,), jnp.float32),
        pltpu.VMEM((RPT,), jnp.float32),
        pltpu.VMEM_SHARED((NPAD,), jnp.float32),
    ],
)
def _sc_degree(dst_hbm, out_hbm, idx_v, ones_v, z_v, acc):
    cid = lax.axis_index("c")
    sid = lax.axis_index("s")
    wid = sid * NC + cid
    pltpu.sync_copy(dst_hbm.at[pl.ds(wid * CPW, CPW)], idx_v)
    for i in range(CH // L):
        ones_v[pl.ds(i * L, L)] = jnp.ones((L,), jnp.float32)

    def zfill(i, c):
        z_v[pl.ds(i * L, L)] = jnp.zeros((L,), jnp.float32)
        return c

    lax.fori_loop(0, RPT // L, zfill, 0)
    pltpu.sync_copy(z_v, acc.at[pl.ds(sid * RPT, RPT)])
    plsc.subcore_barrier()

    def body(j, c):
        pltpu.sync_copy(ones_v, acc.at[idx_v.at[j]], add=True)
        return c

    lax.fori_loop(0, CPW, body, 0)
    plsc.subcore_barrier()
    pltpu.sync_copy(acc.at[pl.ds(sid * RPT, RPT)],
                    out_hbm.at[cid, pl.ds(sid * RPT, RPT)])


D = 128
HCPW = CPW // 2


@functools.partial(
    pl.kernel,
    out_type=jax.ShapeDtypeStruct((NC, NPAD, D), jnp.float32),
    mesh=_MESH,
    scratch_types=[
        pltpu.VMEM((HCPW, CH), jnp.int32),
        pltpu.VMEM((HCPW, CH), jnp.int32),
        pltpu.VMEM((CH, D), jnp.float32),
        pltpu.VMEM((CH, D), jnp.float32),
        pltpu.VMEM_SHARED((NPAD, D), jnp.float32),
        pltpu.SemaphoreType.DMA,
        pltpu.SemaphoreType.DMA,
    ],
)
def _sc_agg(table_hbm, src_hbm, dst_hbm, out_hbm, src_v, dst_v,
            rows_a, rows_b, acc, sem_a, sem_b):
    cid = lax.axis_index("c")
    sid = lax.axis_index("s")
    wid = sid * NC + cid

    def zfill(i, c):
        r = i // (D // L)
        k = i % (D // L)
        rows_a[r, pl.ds(k * L, L)] = jnp.zeros((L,), jnp.float32)
        return c

    with jax.named_scope("agg_zero"):
        lax.fori_loop(0, CH * (D // L), zfill, 0)
        for b in range(RPT // CH):
            pltpu.sync_copy(rows_a, acc.at[pl.ds(sid * RPT + b * CH, CH)])
        plsc.subcore_barrier()

    for h in range(CPW // HCPW):
        with jax.named_scope("agg_idx"):
            base = wid * CPW + h * HCPW
            pltpu.sync_copy(src_hbm.at[pl.ds(base, HCPW)], src_v)
            pltpu.sync_copy(dst_hbm.at[pl.ds(base, HCPW)], dst_v)
        pltpu.async_copy(table_hbm.at[src_v.at[0]], rows_a, sem_a)
        pltpu.async_copy(table_hbm.at[src_v.at[1]], rows_b, sem_b)

        def body(i, c):
            ja = 2 * i
            jb = ja + 1
            pltpu.make_async_copy(table_hbm, rows_a, sem_a).wait()
            pltpu.sync_copy(rows_a, acc.at[dst_v.at[ja]], add=True)

            @pl.when(ja + 2 < HCPW)
            def _():
                pltpu.async_copy(table_hbm.at[src_v.at[ja + 2]], rows_a, sem_a)

            pltpu.make_async_copy(table_hbm, rows_b, sem_b).wait()
            pltpu.sync_copy(rows_b, acc.at[dst_v.at[jb]], add=True)

            @pl.when(jb + 2 < HCPW)
            def _():
                pltpu.async_copy(table_hbm.at[src_v.at[jb + 2]], rows_b, sem_b)

            return c

        with jax.named_scope("agg_loop"):
            lax.fori_loop(0, HCPW // 2, body, 0)

    with jax.named_scope("agg_dump"):
        plsc.subcore_barrier()
        pltpu.sync_copy(acc.at[pl.ds(sid * RPT, RPT)],
                        out_hbm.at[cid, pl.ds(sid * RPT, RPT)])


def _tc_prescale_body(x_ref, w_ref, d0_ref, d1_ref, hs_ref, dinv_ref):
    deg = d0_ref[...] + d1_ref[...] + 1.0
    dinv = lax.rsqrt(deg)
    dinv_ref[...] = dinv
    h = jnp.dot(x_ref[...], w_ref[...], preferred_element_type=jnp.float32)
    hs_ref[0:N, 0:C1] = h * dinv
    hs_ref[0:N, C1:D] = jnp.zeros((N, D - C1), jnp.float32)
    hs_ref[N:TPAD, :] = jnp.zeros((TPAD - N, D), jnp.float32)


def _tc_layer2_body(a0_ref, a1_ref, hs_ref, dinv_ref, b1_ref, w2_ref,
                    h2s_ref):
    dinv = dinv_ref[...]
    pre = dinv * (a0_ref[...] + a1_ref[...] + hs_ref[0:N, 0:C1]) + b1_ref[...]
    h1 = jnp.maximum(pre, 0.0)
    h2 = jnp.dot(h1, w2_ref[...], preferred_element_type=jnp.float32)
    h2s_ref[0:N, :] = h2 * dinv
    h2s_ref[N:TPAD, :] = jnp.zeros((TPAD - N, C2), jnp.float32)


def _tc_head_body(a0_ref, a1_ref, hs_ref, dinv_ref, b2_ref, batch_ref,
                  fcw_ref, fcb_ref, out_ref):
    dinv = dinv_ref[...]
    pre = dinv * (a0_ref[...] + a1_ref[...] + hs_ref[0:N, :]) + b2_ref[...]
    h2 = jnp.maximum(pre, 0.0)
    gid = lax.broadcasted_iota(jnp.int32, (N, NG), 1)
    p = (batch_ref[...] == gid).astype(jnp.float32)
    sums = lax.dot_general(p, h2, (((0,), (0,)), ((), ())),
                           preferred_element_type=jnp.float32)
    ones = jnp.ones((N, 1), jnp.float32)
    cnt = lax.dot_general(p, ones, (((0,), (0,)), ((), ())),
                          preferred_element_type=jnp.float32)
    g = sums / jnp.maximum(cnt, 1.0)
    logits = jnp.dot(g, fcw_ref[...], preferred_element_type=jnp.float32)
    logits = logits + fcb_ref[...]
    m = jnp.max(logits, axis=1, keepdims=True)
    ex = jnp.exp(logits - m)
    lse = jnp.log(jnp.sum(ex, axis=1, keepdims=True))
    out_ref[...] = logits - m - lse


_tc_prescale = pl.pallas_call(
    _tc_prescale_body,
    out_shape=(jax.ShapeDtypeStruct((TPAD, D), jnp.float32),
               jax.ShapeDtypeStruct((N, 1), jnp.float32)))

_tc_layer2 = pl.pallas_call(
    _tc_layer2_body,
    out_shape=jax.ShapeDtypeStruct((TPAD, C2), jnp.float32))

_tc_head = pl.pallas_call(
    _tc_head_body,
    out_shape=jax.ShapeDtypeStruct((NG, NCLS), jnp.float32))


def kernel(x, edge_index, batch, W1, b1, W2, b2, fc_W, fc_b):
    src = edge_index[0].astype(jnp.int32)
    dst = edge_index[1].astype(jnp.int32)
    it = jnp.arange(E_PAD - E, dtype=jnp.int32)
    pad_src = it % N
    pad_dst = N + it % (NPAD - N)
    src2d = jnp.concatenate([src, pad_src]).reshape(NW * CPW, CH)
    dst2d = jnp.concatenate([dst, pad_dst]).reshape(NW * CPW, CH)
    batch2d = batch.astype(jnp.int32).reshape(N, 1)

    degp = _sc_degree(dst2d)
    d0 = degp[0, :N].reshape(N, 1)
    d1 = degp[1, :N].reshape(N, 1)

    h1s, dinv = _tc_prescale(x, W1, d0, d1)

    agg1 = _sc_agg(h1s, src2d, dst2d)
    h2s = _tc_layer2(agg1[0, :N, :C1], agg1[1, :N, :C1], h1s, dinv,
                     b1.reshape(1, C1), W2)

    agg2 = _sc_agg(h2s, src2d, dst2d)
    out = _tc_head(agg2[0, :N, :], agg2[1, :N, :], h2s, dinv,
                   b2.reshape(1, C2), batch2d,
                   fc_W, fc_b.reshape(1, NCLS))
    return out


# speedup vs baseline: 3.7229x; 1.0485x over previous
import functools

import jax
import jax.numpy as jnp
from jax import lax
from jax.experimental import pallas as pl
from jax.experimental.pallas import tpu as pltpu
from jax.experimental.pallas import tpu_sc as plsc

N = 10000
E = 320000
DF = 128
C1 = 64
C2 = 128
NCLS = 10
NG = 32

NC = 2
NS = 16
L = 16
NW = NC * NS

CH = 128
CPW = 80
EPW = CH * CPW
E_PAD = EPW * NW
NPAD = 10240
RPT = NPAD // NS
TPAD = 10048

_MESH = plsc.VectorSubcoreMesh(core_axis_name="c", subcore_axis_name="s")


@functools.partial(
    pl.kernel,
    out_type=jax.ShapeDtypeStruct((NC, NPAD), jnp.float32),
    mesh=_MESH,
    scratch_types=[
        pltpu.VMEM((CPW, CH), jnp.int32),
        pltpu.VMEM((CH,), jnp.float32),
        pltpu.VMEM((RPT,), jnp.float32),
        pltpu.VMEM_SHARED((NPAD,), jnp.float32),
    ],
)
def _sc_degree(dst_hbm, out_hbm, idx_v, ones_v, z_v, acc):
    cid = lax.axis_index("c")
    sid = lax.axis_index("s")
    wid = sid * NC + cid
    pltpu.sync_copy(dst_hbm.at[pl.ds(wid * CPW, CPW)], idx_v)
    for i in range(CH // L):
        ones_v[pl.ds(i * L, L)] = jnp.ones((L,), jnp.float32)

    def zfill(i, c):
        z_v[pl.ds(i * L, L)] = jnp.zeros((L,), jnp.float32)
        return c

    lax.fori_loop(0, RPT // L, zfill, 0)
    pltpu.sync_copy(z_v, acc.at[pl.ds(sid * RPT, RPT)])
    plsc.subcore_barrier()

    def body(j, c):
        pltpu.sync_copy(ones_v, acc.at[idx_v.at[j]], add=True)
        return c

    lax.fori_loop(0, CPW, body, 0)
    plsc.subcore_barrier()
    pltpu.sync_copy(acc.at[pl.ds(sid * RPT, RPT)],
                    out_hbm.at[cid, pl.ds(sid * RPT, RPT)])


D = 128
HCPW = CPW // 2


@functools.partial(
    pl.kernel,
    out_type=jax.ShapeDtypeStruct((NC, NPAD, D), jnp.float32),
    mesh=_MESH,
    scratch_types=[
        pltpu.VMEM((HCPW, CH), jnp.int32),
        pltpu.VMEM((HCPW, CH), jnp.int32),
        pltpu.VMEM((CH, D), jnp.float32),
        pltpu.VMEM((CH, D), jnp.float32),
        pltpu.VMEM_SHARED((NPAD, D), jnp.float32),
        pltpu.SemaphoreType.DMA,
        pltpu.SemaphoreType.DMA,
    ],
)
def _sc_agg(table_hbm, src_hbm, dst_hbm, out_hbm, src_v, dst_v,
            rows_a, rows_b, acc, sem_a, sem_b):
    cid = lax.axis_index("c")
    sid = lax.axis_index("s")
    wid = sid * NC + cid

    def zfill(i, c):
        r = i // (D // L)
        k = i % (D // L)
        rows_a[r, pl.ds(k * L, L)] = jnp.zeros((L,), jnp.float32)
        return c

    with jax.named_scope("agg_zero"):
        lax.fori_loop(0, CH * (D // L), zfill, 0)
        for b in range(RPT // CH):
            pltpu.sync_copy(rows_a, acc.at[pl.ds(sid * RPT + b * CH, CH)])
        plsc.subcore_barrier()

    for h in range(CPW // HCPW):
        with jax.named_scope("agg_idx"):
            base = wid * CPW + h * HCPW
            pltpu.sync_copy(src_hbm.at[pl.ds(base, HCPW)], src_v)
            pltpu.sync_copy(dst_hbm.at[pl.ds(base, HCPW)], dst_v)
        pltpu.async_copy(table_hbm.at[src_v.at[0]], rows_a, sem_a)
        pltpu.async_copy(table_hbm.at[src_v.at[1]], rows_b, sem_b)

        def body(i, c):
            ja = 2 * i
            jb = ja + 1
            pltpu.make_async_copy(table_hbm, rows_a, sem_a).wait()
            pltpu.sync_copy(rows_a, acc.at[dst_v.at[ja]], add=True)

            @pl.when(ja + 2 < HCPW)
            def _():
                pltpu.async_copy(table_hbm.at[src_v.at[ja + 2]], rows_a, sem_a)

            pltpu.make_async_copy(table_hbm, rows_b, sem_b).wait()
            pltpu.sync_copy(rows_b, acc.at[dst_v.at[jb]], add=True)

            @pl.when(jb + 2 < HCPW)
            def _():
                pltpu.async_copy(table_hbm.at[src_v.at[jb + 2]], rows_b, sem_b)

            return c

        with jax.named_scope("agg_loop"):
            lax.fori_loop(0, HCPW // 2, body, 0)

    with jax.named_scope("agg_dump"):
        plsc.subcore_barrier()
        pltpu.sync_copy(acc.at[pl.ds(sid * RPT, RPT)],
                        out_hbm.at[cid, pl.ds(sid * RPT, RPT)])


def _tc_prescale_body(x_ref, w_ref, degp_ref, hs_ref, dinv_ref):
    deg = degp_ref[0, 0:N, :] + degp_ref[1, 0:N, :] + 1.0
    dinv = lax.rsqrt(deg)
    dinv_ref[...] = dinv
    h = jnp.dot(x_ref[...], w_ref[...], preferred_element_type=jnp.float32)
    hs_ref[0:N, 0:C1] = h * dinv
    hs_ref[0:N, C1:D] = jnp.zeros((N, D - C1), jnp.float32)
    hs_ref[N:TPAD, :] = jnp.zeros((TPAD - N, D), jnp.float32)


def _tc_layer2_body(agg_ref, hs_ref, dinv_ref, b1_ref, w2_ref,
                    h2s_ref):
    dinv = dinv_ref[...]
    a0 = agg_ref[0, 0:N, 0:C1]
    a1 = agg_ref[1, 0:N, 0:C1]
    pre = dinv * (a0 + a1 + hs_ref[0:N, 0:C1]) + b1_ref[...]
    h1 = jnp.maximum(pre, 0.0)
    h2 = jnp.dot(h1, w2_ref[...], preferred_element_type=jnp.float32)
    h2s_ref[0:N, :] = h2 * dinv
    h2s_ref[N:TPAD, :] = jnp.zeros((TPAD - N, C2), jnp.float32)


def _tc_head_body(agg_ref, hs_ref, dinv_ref, b2_ref, batch_ref,
                  fcw_ref, fcb_ref, out_ref):
    dinv = dinv_ref[...]
    a0 = agg_ref[0, 0:N, :]
    a1 = agg_ref[1, 0:N, :]
    pre = dinv * (a0 + a1 + hs_ref[0:N, :]) + b2_ref[...]
    h2 = jnp.maximum(pre, 0.0)
    gid = lax.broadcasted_iota(jnp.int32, (N, NG), 1)
    p = (batch_ref[...] == gid).astype(jnp.float32)
    sums = lax.dot_general(p, h2, (((0,), (0,)), ((), ())),
                           preferred_element_type=jnp.float32)
    ones = jnp.ones((N, 1), jnp.float32)
    cnt = lax.dot_general(p, ones, (((0,), (0,)), ((), ())),
                          preferred_element_type=jnp.float32)
    g = sums / jnp.maximum(cnt, 1.0)
    logits = jnp.dot(g, fcw_ref[...], preferred_element_type=jnp.float32)
    logits = logits + fcb_ref[...]
    m = jnp.max(logits, axis=1, keepdims=True)
    ex = jnp.exp(logits - m)
    lse = jnp.log(jnp.sum(ex, axis=1, keepdims=True))
    out_ref[...] = logits - m - lse


_tc_prescale = pl.pallas_call(
    _tc_prescale_body,
    out_shape=(jax.ShapeDtypeStruct((TPAD, D), jnp.float32),
               jax.ShapeDtypeStruct((N, 1), jnp.float32)))

_tc_layer2 = pl.pallas_call(
    _tc_layer2_body,
    out_shape=jax.ShapeDtypeStruct((TPAD, C2), jnp.float32))

_tc_head = pl.pallas_call(
    _tc_head_body,
    out_shape=jax.ShapeDtypeStruct((NG, NCLS), jnp.float32))


def kernel(x, edge_index, batch, W1, b1, W2, b2, fc_W, fc_b):
    src = edge_index[0].astype(jnp.int32)
    dst = edge_index[1].astype(jnp.int32)
    it = jnp.arange(E_PAD - E, dtype=jnp.int32)
    pad_src = it % N
    pad_dst = N + it % (NPAD - N)
    src2d = jnp.concatenate([src, pad_src]).reshape(NW * CPW, CH)
    dst2d = jnp.concatenate([dst, pad_dst]).reshape(NW * CPW, CH)
    batch2d = batch.astype(jnp.int32).reshape(N, 1)

    degp = _sc_degree(dst2d).reshape(NC, NPAD, 1)

    h1s, dinv = _tc_prescale(x, W1, degp)

    agg1 = _sc_agg(h1s, src2d, dst2d)
    h2s = _tc_layer2(agg1, h1s, dinv, b1.reshape(1, C1), W2)

    agg2 = _sc_agg(h2s, src2d, dst2d)
    out = _tc_head(agg2, h2s, dinv,
                   b2.reshape(1, C2), batch2d,
                   fc_W, fc_b.reshape(1, NCLS))
    return out


# speedup vs baseline: 4.1121x; 1.1045x over previous
import functools

import jax
import jax.numpy as jnp
from jax import lax
from jax.experimental import pallas as pl
from jax.experimental.pallas import tpu as pltpu
from jax.experimental.pallas import tpu_sc as plsc

N = 10000
E = 320000
DF = 128
C1 = 64
C2 = 128
NCLS = 10
NG = 32

NC = 2
NS = 16
L = 16
NW = NC * NS

CH = 128
CPW = 80
EPW = CH * CPW
E_PAD = EPW * NW
NPAD = 10240
RPT = NPAD // NS
TPAD = 10048

_MESH = plsc.VectorSubcoreMesh(core_axis_name="c", subcore_axis_name="s")


@functools.partial(
    pl.kernel,
    out_type=jax.ShapeDtypeStruct((NC, NPAD), jnp.float32),
    mesh=_MESH,
    scratch_types=[
        pltpu.VMEM((CPW, CH), jnp.int32),
        pltpu.VMEM((CH,), jnp.float32),
        pltpu.VMEM((RPT,), jnp.float32),
        pltpu.VMEM_SHARED((NPAD,), jnp.float32),
    ],
)
def _sc_degree(dst_hbm, out_hbm, idx_v, ones_v, z_v, acc):
    cid = lax.axis_index("c")
    sid = lax.axis_index("s")
    wid = sid * NC + cid
    pltpu.sync_copy(dst_hbm.at[pl.ds(wid * CPW, CPW)], idx_v)
    for i in range(CH // L):
        ones_v[pl.ds(i * L, L)] = jnp.ones((L,), jnp.float32)

    def zfill(i, c):
        z_v[pl.ds(i * L, L)] = jnp.zeros((L,), jnp.float32)
        return c

    lax.fori_loop(0, RPT // L, zfill, 0)
    pltpu.sync_copy(z_v, acc.at[pl.ds(sid * RPT, RPT)])
    plsc.subcore_barrier()

    def body(j, c):
        pltpu.sync_copy(ones_v, acc.at[idx_v.at[j]], add=True)
        return c

    lax.fori_loop(0, CPW, body, 0)
    plsc.subcore_barrier()
    pltpu.sync_copy(acc.at[pl.ds(sid * RPT, RPT)],
                    out_hbm.at[cid, pl.ds(sid * RPT, RPT)])


D = 128
HCPW = CPW // 2


@functools.partial(
    pl.kernel,
    out_type=jax.ShapeDtypeStruct((NC, NPAD, D), jnp.float32),
    mesh=_MESH,
    scratch_types=[
        pltpu.VMEM((HCPW, CH), jnp.int32),
        pltpu.VMEM((HCPW, CH), jnp.int32),
        pltpu.VMEM((CH, D), jnp.float32),
        pltpu.VMEM((CH, D), jnp.float32),
        pltpu.VMEM_SHARED((NPAD, D), jnp.float32),
        pltpu.SemaphoreType.DMA,
        pltpu.SemaphoreType.DMA,
    ],
)
def _sc_agg(table_hbm, src_hbm, dst_hbm, out_hbm, src_v, dst_v,
            rows_a, rows_b, acc, sem_a, sem_b):
    cid = lax.axis_index("c")
    sid = lax.axis_index("s")
    wid = sid * NC + cid

    def zfill(i, c):
        r = i // (D // L)
        k = i % (D // L)
        rows_a[r, pl.ds(k * L, L)] = jnp.zeros((L,), jnp.float32)
        return c

    with jax.named_scope("agg_zero"):
        lax.fori_loop(0, CH * (D // L), zfill, 0)
        for b in range(RPT // CH):
            pltpu.sync_copy(rows_a, acc.at[pl.ds(sid * RPT + b * CH, CH)])
        plsc.subcore_barrier()

    for h in range(CPW // HCPW):
        with jax.named_scope("agg_idx"):
            base = wid * CPW + h * HCPW
            pltpu.sync_copy(src_hbm.at[pl.ds(base, HCPW)], src_v)
            pltpu.sync_copy(dst_hbm.at[pl.ds(base, HCPW)], dst_v)
        pltpu.async_copy(table_hbm.at[src_v.at[0]], rows_a, sem_a)
        pltpu.async_copy(table_hbm.at[src_v.at[1]], rows_b, sem_b)

        def body(i, c):
            ja = 2 * i
            jb = ja + 1
            pltpu.make_async_copy(table_hbm, rows_a, sem_a).wait()
            pltpu.sync_copy(rows_a, acc.at[dst_v.at[ja]], add=True)

            @pl.when(ja + 2 < HCPW)
            def _():
                pltpu.async_copy(table_hbm.at[src_v.at[ja + 2]], rows_a, sem_a)

            pltpu.make_async_copy(table_hbm, rows_b, sem_b).wait()
            pltpu.sync_copy(rows_b, acc.at[dst_v.at[jb]], add=True)

            @pl.when(jb + 2 < HCPW)
            def _():
                pltpu.async_copy(table_hbm.at[src_v.at[jb + 2]], rows_b, sem_b)

            return c

        with jax.named_scope("agg_loop"):
            lax.fori_loop(0, HCPW // 2, body, 0)

    with jax.named_scope("agg_dump"):
        plsc.subcore_barrier()
        pltpu.sync_copy(acc.at[pl.ds(sid * RPT, RPT)],
                        out_hbm.at[cid, pl.ds(sid * RPT, RPT)])


@functools.partial(
    pl.kernel,
    out_type=jax.ShapeDtypeStruct((NC, NPAD, C1), jnp.float32),
    mesh=_MESH,
    compiler_params=pltpu.CompilerParams(use_tc_tiling_on_sc=False),
    scratch_types=[
        pltpu.VMEM((CPW, CH), jnp.int32),
        pltpu.VMEM((CPW, CH), jnp.int32),
        pltpu.VMEM((CH, C1), jnp.float32),
        pltpu.VMEM((CH, C1), jnp.float32),
        pltpu.VMEM_SHARED((NPAD, C1), jnp.float32),
        pltpu.SemaphoreType.DMA,
        pltpu.SemaphoreType.DMA,
    ],
)
def _sc_agg64(table_hbm, src_hbm, dst_hbm, out_hbm, src_v, dst_v,
              rows_a, rows_b, acc, sem_a, sem_b):
    cid = lax.axis_index("c")
    sid = lax.axis_index("s")
    wid = sid * NC + cid
    pltpu.sync_copy(src_hbm.at[pl.ds(wid * CPW, CPW)], src_v)
    pltpu.sync_copy(dst_hbm.at[pl.ds(wid * CPW, CPW)], dst_v)

    def zfill(i, c):
        r = i // (C1 // L)
        k = i % (C1 // L)
        rows_a[r, pl.ds(k * L, L)] = jnp.zeros((L,), jnp.float32)
        return c

    lax.fori_loop(0, CH * (C1 // L), zfill, 0)
    for b in range(RPT // CH):
        pltpu.sync_copy(rows_a, acc.at[pl.ds(sid * RPT + b * CH, CH)])
    plsc.subcore_barrier()

    pltpu.async_copy(table_hbm.at[src_v.at[0]], rows_a, sem_a)
    pltpu.async_copy(table_hbm.at[src_v.at[1]], rows_b, sem_b)

    def body(i, c):
        ja = 2 * i
        jb = ja + 1
        pltpu.make_async_copy(table_hbm, rows_a, sem_a).wait()
        pltpu.sync_copy(rows_a, acc.at[dst_v.at[ja]], add=True)

        @pl.when(ja + 2 < CPW)
        def _():
            pltpu.async_copy(table_hbm.at[src_v.at[ja + 2]], rows_a, sem_a)

        pltpu.make_async_copy(table_hbm, rows_b, sem_b).wait()
        pltpu.sync_copy(rows_b, acc.at[dst_v.at[jb]], add=True)

        @pl.when(jb + 2 < CPW)
        def _():
            pltpu.async_copy(table_hbm.at[src_v.at[jb + 2]], rows_b, sem_b)

        return c

    lax.fori_loop(0, CPW // 2, body, 0)
    plsc.subcore_barrier()
    pltpu.sync_copy(acc.at[pl.ds(sid * RPT, RPT)],
                    out_hbm.at[cid, pl.ds(sid * RPT, RPT)])


def _tc_prescale_body(x_ref, w_ref, degp_ref, hs_ref, dinv_ref):
    deg = degp_ref[0, 0:N, :] + degp_ref[1, 0:N, :] + 1.0
    dinv = lax.rsqrt(deg)
    dinv_ref[...] = dinv
    h = jnp.dot(x_ref[...], w_ref[...], preferred_element_type=jnp.float32)
    hs_ref[0:N, :] = h * dinv
    hs_ref[N:TPAD, :] = jnp.zeros((TPAD - N, C1), jnp.float32)


def _tc_layer2_body(agg_ref, hs_ref, dinv_ref, b1_ref, w2_ref,
                    h2s_ref):
    dinv = dinv_ref[...]
    a0 = agg_ref[0, 0:N, :]
    a1 = agg_ref[1, 0:N, :]
    pre = dinv * (a0 + a1 + hs_ref[0:N, :]) + b1_ref[...]
    h1 = jnp.maximum(pre, 0.0)
    h2 = jnp.dot(h1, w2_ref[...], preferred_element_type=jnp.float32)
    h2s_ref[0:N, :] = h2 * dinv
    h2s_ref[N:TPAD, :] = jnp.zeros((TPAD - N, C2), jnp.float32)


def _tc_head_body(agg_ref, hs_ref, dinv_ref, b2_ref, batch_ref,
                  fcw_ref, fcb_ref, out_ref):
    dinv = dinv_ref[...]
    a0 = agg_ref[0, 0:N, :]
    a1 = agg_ref[1, 0:N, :]
    pre = dinv * (a0 + a1 + hs_ref[0:N, :]) + b2_ref[...]
    h2 = jnp.maximum(pre, 0.0)
    gid = lax.broadcasted_iota(jnp.int32, (N, NG), 1)
    p = (batch_ref[...] == gid).astype(jnp.float32)
    sums = lax.dot_general(p, h2, (((0,), (0,)), ((), ())),
                           preferred_element_type=jnp.float32)
    ones = jnp.ones((N, 1), jnp.float32)
    cnt = lax.dot_general(p, ones, (((0,), (0,)), ((), ())),
                          preferred_element_type=jnp.float32)
    g = sums / jnp.maximum(cnt, 1.0)
    logits = jnp.dot(g, fcw_ref[...], preferred_element_type=jnp.float32)
    logits = logits + fcb_ref[...]
    m = jnp.max(logits, axis=1, keepdims=True)
    ex = jnp.exp(logits - m)
    lse = jnp.log(jnp.sum(ex, axis=1, keepdims=True))
    out_ref[...] = logits - m - lse


_tc_prescale = pl.pallas_call(
    _tc_prescale_body,
    out_shape=(jax.ShapeDtypeStruct((TPAD, C1), jnp.float32),
               jax.ShapeDtypeStruct((N, 1), jnp.float32)))

_tc_layer2 = pl.pallas_call(
    _tc_layer2_body,
    out_shape=jax.ShapeDtypeStruct((TPAD, C2), jnp.float32))

_tc_head = pl.pallas_call(
    _tc_head_body,
    out_shape=jax.ShapeDtypeStruct((NG, NCLS), jnp.float32))


def kernel(x, edge_index, batch, W1, b1, W2, b2, fc_W, fc_b):
    src = edge_index[0].astype(jnp.int32)
    dst = edge_index[1].astype(jnp.int32)
    it = jnp.arange(E_PAD - E, dtype=jnp.int32)
    pad_src = it % N
    pad_dst = N + it % (NPAD - N)
    src2d = jnp.concatenate([src, pad_src]).reshape(NW * CPW, CH)
    dst2d = jnp.concatenate([dst, pad_dst]).reshape(NW * CPW, CH)
    batch2d = batch.astype(jnp.int32).reshape(N, 1)

    degp = _sc_degree(dst2d).reshape(NC, NPAD, 1)

    h1s, dinv = _tc_prescale(x, W1, degp)

    agg1 = _sc_agg64(h1s, src2d, dst2d)
    h2s = _tc_layer2(agg1, h1s, dinv, b1.reshape(1, C1), W2)

    agg2 = _sc_agg(h2s, src2d, dst2d)
    out = _tc_head(agg2, h2s, dinv,
                   b2.reshape(1, C2), batch2d,
                   fc_W, fc_b.reshape(1, NCLS))
    return out
